# baseline (speedup 1.0000x reference)
import functools

import jax
import jax.numpy as jnp
from jax import lax
from jax.experimental import pallas as pl
from jax.experimental.pallas import tpu as pltpu
from jax.experimental.pallas import tpu_sc as plsc

_D = 80
_HE = 64
_SAW = 128
_TILE = 4096
_IDXC = 128


def _sc_gather(table, idx):
    _, d = table.shape
    b = idx.shape[0]
    info = plsc.get_sparse_core_info()
    nw = info.num_cores * info.num_subcores
    b_per_w = b // nw
    nchunk = b_per_w // _IDXC
    mesh = plsc.VectorSubcoreMesh(core_axis_name="c", subcore_axis_name="s")

    @functools.partial(
        pl.kernel,
        mesh=mesh,
        out_type=jax.ShapeDtypeStruct((b, d), jnp.float32),
        scratch_types=[
            pltpu.VMEM((nchunk, _IDXC), jnp.int32),
            pltpu.VMEM((nchunk, _IDXC, d), jnp.float32),
            pltpu.SemaphoreType.DMA,
        ],
        compiler_params=pltpu.CompilerParams(use_tc_tiling_on_sc=False),
    )
    def k(table_hbm, idx_hbm, out_hbm, idx_v, rows_v, sem):
        wid = lax.axis_index("s") * info.num_cores + lax.axis_index("c")
        base = wid * b_per_w
        for j in range(nchunk):
            pltpu.sync_copy(idx_hbm.at[pl.ds(base + j * _IDXC, _IDXC)],
                            idx_v.at[j])
        copies = [
            pltpu.async_copy(table_hbm.at[idx_v.at[j]], rows_v.at[j], sem)
            for j in range(nchunk)
        ]
        for c in copies:
            c.wait()
        for j in range(nchunk):
            pltpu.sync_copy(rows_v.at[j],
                            out_hbm.at[pl.ds(base + j * _IDXC, _IDXC)])

    return k(table, idx)


def _tc_body(z_ref, tv_ref, ball_ref,
             ew1_ref, wn1_ref, wr1_ref, wr2_ref, ew2_ref, wn2_ref, wn3_ref,
             ww1_ref, ww2_ref, ww3_ref, wi1_ref, wt1_ref, wi2_ref, wt2_ref,
             wi3_ref, wt3_ref, wo1_ref, wo2_ref, wo3_ref,
             hout_ref, cur_ref,
             wp1_s, wp2_s, wd_s, we_s, wf_s,
             wa_s, wb_s, wc_s, wn3_s, wg_s, wh_s, wi_s,
             bp1_s, bp2_s, bn3_s, ba_s, bb_s, bc_s, bd_s, be_s, bf_s,
             bg_s, bh_s, bi_s):
    f32 = jnp.float32

    bf16 = jnp.bfloat16

    @pl.when(pl.program_id(0) == 0)
    def _prep():
        bcol = jnp.transpose(ball_ref[...])

        wp1_s[0:64, :] = ew1_ref[0].astype(bf16)
        wp1_s[64:128, :] = ew1_ref[1].astype(bf16)
        wp1_s[128:208, :] = wn1_ref[...].astype(bf16)
        wp1_s[208:210, :] = wr1_ref[0:2, :].astype(bf16)
        wp1_s[210:212, :] = wr2_ref[0:2, :].astype(bf16)
        bp1_s[0:128, :] = bcol[0:128]
        bp1_s[128:208, :] = bcol[128:208]
        bp1_s[208:210, :] = bcol[208:210]
        bp1_s[210:212, :] = bcol[272:274]

        wp2_s[...] = jnp.zeros(wp2_s.shape, bf16)
        wp2_s[0:80, 0:64] = ew2_ref[0].astype(bf16)
        wp2_s[80:160, 64:128] = ew2_ref[1].astype(bf16)
        wp2_s[160:200, 128:208] = wn2_ref[...].astype(bf16)
        bp2_s[0:160, :] = bcol[336:496]
        bp2_s[160:200, :] = bcol[496:536]
        bn3_s[...] = bcol[1720:1721]

        ba_s[...] = bcol[536:664]
        bb_s[...] = bcol[664:792]
        bc_s[...] = bcol[792:856]

        wd_s[0:128, :] = wi1_ref[...].astype(bf16)
        wd_s[128:256, :] = wt1_ref[...].astype(bf16)
        bd_s[0:128, :] = bcol[856:984]
        bd_s[128:256, :] = bcol[984:1112]

        we_s[...] = jnp.zeros(we_s.shape, bf16)
        we_s[0:64, 0:128] = wi2_ref[...].astype(bf16)
        we_s[64:192, 128:256] = wt2_ref[...].astype(bf16)
        be_s[0:64, :] = bcol[1112:1176]
        be_s[64:192, :] = bcol[1176:1304]

        wf_s[...] = jnp.zeros(wf_s.shape, bf16)
        wf_s[0:80, 64:192] = wt3_ref[...].astype(bf16)
        wf_s[80:81, 0:64] = wi3_ref[...].astype(bf16)
        bf_s[0:80, :] = bcol[1304:1384]
        bf_s[80:81, :] = bcol[1721:1722]

        bg_s[...] = bcol[1384:1512]
        bh_s[...] = bcol[1512:1640]
        bi_s[...] = bcol[1640:1720]
        wa_s[...] = ww1_ref[...].astype(bf16)
        wb_s[...] = ww2_ref[...].astype(bf16)
        wc_s[...] = ww3_ref[...].astype(bf16)
        wn3_s[...] = wn3_ref[...].astype(bf16)
        wg_s[...] = wo1_ref[...].astype(bf16)
        wh_s[...] = wo2_ref[...].astype(bf16)
        wi_s[...] = wo3_ref[...].astype(bf16)

    def mm(w, b_ref, x):
        y = lax.dot_general(w, x.astype(jnp.bfloat16),
                            (((1,), (0,)), ((), ())),
                            preferred_element_type=f32)
        return y + b_ref[...]

    sig = jax.nn.sigmoid
    n = z_ref.shape[0]
    h = jnp.concatenate(
        [jnp.transpose(z_ref[...]), jnp.transpose(tv_ref[...])], axis=0)
    cur_acc = jnp.zeros((1, n), f32)
    ctx_acc = jnp.zeros((1, n), f32)
    for _ in range(2):
        p1 = mm(wp1_s[...], bp1_s, h)
        r = jnp.maximum(p1[:208], 0.0)
        p2 = jnp.maximum(mm(wp2_s[...], bp2_s, r), 0.0)
        novelty = sig(mm(wn3_s[...], bn3_s, p2[160:200]))
        d1 = p1[208:209] - p1[209:210]
        d2 = p1[210:211] - p1[211:212]
        m0 = (sig(d1) + sig(d2)) * 0.5
        m1 = (sig(-d1) + sig(-d2)) * 0.5
        h_next = p2[:80] * m0 + p2[80:160] * m1
        diff = h_next - h
        fe = jnp.sqrt(jnp.sum(diff * diff, axis=0, keepdims=True))
        err = sig(fe - 0.5)
        cur_acc = cur_acc + jnp.clip(0.4 * err + 0.6 * novelty, 0.1, 0.9)
        ctx_acc = ctx_acc + (m0 + m1)
        h = h_next
    ctx = ctx_acc * 0.25

    w = jnp.maximum(mm(wa_s[...], ba_s,
                       jnp.concatenate([h, ctx * h], axis=0)), 0.0)
    w = jnp.maximum(mm(wb_s[...], bb_s, w), 0.0)
    why = mm(wc_s[...], bc_s, w)
    qt = jnp.maximum(mm(wd_s[...], bd_s,
                        jnp.concatenate([h, why], axis=0)), 0.0)
    qt = jnp.maximum(mm(we_s[...], be_s, qt), 0.0)
    f = mm(wf_s[...], bf_s, qt)
    blended = h + sig(f[80:81]) * f[:80]
    o = jnp.maximum(mm(wg_s[...], bg_s,
                       jnp.concatenate([h, blended], axis=0)), 0.0)
    o = jnp.maximum(mm(wh_s[...], bh_s, o), 0.0)
    h_final = 0.7 * h + 0.3 * mm(wi_s[...], bi_s, o)
    hout_ref[...] = jnp.transpose(h_final)
    cur_ref[...] = jnp.transpose(cur_acc * 0.5)


def _tc_forward(z, tv, ball, raw):
    b = z.shape[0]
    grid = (b // _TILE,)
    f32 = jnp.float32

    def rowspec(cols):
        return pl.BlockSpec((_TILE, cols), lambda i: (i, 0))

    def fullspec(arr):
        return pl.BlockSpec(arr.shape, lambda i: (0,) * arr.ndim)

    in_specs = [rowspec(z.shape[1]), rowspec(tv.shape[1]), fullspec(ball)]
    in_specs += [fullspec(x) for x in raw]
    return pl.pallas_call(
        _tc_body,
        grid=grid,
        in_specs=in_specs,
        out_specs=[rowspec(_D), rowspec(1)],
        out_shape=[
            jax.ShapeDtypeStruct((b, _D), f32),
            jax.ShapeDtypeStruct((b, 1), f32),
        ],
        scratch_shapes=[
            pltpu.VMEM((212, 80), jnp.bfloat16),
            pltpu.VMEM((200, 208), jnp.bfloat16),
            pltpu.VMEM((256, 144), jnp.bfloat16),
            pltpu.VMEM((192, 256), jnp.bfloat16),
            pltpu.VMEM((81, 192), jnp.bfloat16),
            pltpu.VMEM((128, 160), jnp.bfloat16),
            pltpu.VMEM((128, 128), jnp.bfloat16),
            pltpu.VMEM((64, 128), jnp.bfloat16),
            pltpu.VMEM((1, 40), jnp.bfloat16),
            pltpu.VMEM((128, 160), jnp.bfloat16),
            pltpu.VMEM((128, 128), jnp.bfloat16),
            pltpu.VMEM((80, 128), jnp.bfloat16),
            pltpu.VMEM((212, 1), f32), pltpu.VMEM((200, 1), f32),
            pltpu.VMEM((1, 1), f32), pltpu.VMEM((128, 1), f32),
            pltpu.VMEM((128, 1), f32), pltpu.VMEM((64, 1), f32),
            pltpu.VMEM((256, 1), f32), pltpu.VMEM((192, 1), f32),
            pltpu.VMEM((81, 1), f32), pltpu.VMEM((128, 1), f32),
            pltpu.VMEM((128, 1), f32), pltpu.VMEM((80, 1), f32),
        ],
        compiler_params=pltpu.CompilerParams(
            dimension_semantics=("arbitrary",),
        ),
    )(z, tv, ball, *raw)


def kernel(z, task_ids, params):
    p = params
    tv = _sc_gather(p['task_emb'].astype(jnp.float32),
                    task_ids.astype(jnp.int32))
    ball = jnp.concatenate(
        [p[k].reshape(1, -1).astype(jnp.float32) for k in
         ('eb1', 'bn1', 'br1', 'br2', 'eb2', 'bn2', 'bw1', 'bw2', 'bw3',
          'bi1', 'bt1', 'bi2', 'bt2', 'bt3', 'bo1', 'bo2', 'bo3', 'bn3',
          'bi3')], axis=1)
    raw = [p[k] for k in
           ('eW1', 'Wn1', 'Wr1', 'Wr2', 'eW2', 'Wn2', 'Wn3', 'Ww1', 'Ww2',
            'Ww3', 'Wi1', 'Wt1', 'Wi2', 'Wt2', 'Wi3', 'Wt3', 'Wo1', 'Wo2',
            'Wo3')]
    h_final, avg_cur = _tc_forward(z, tv, ball, raw)
    return h_final, avg_cur


# speedup vs baseline: 1.0046x; 1.0046x over previous
import functools

import jax
import jax.numpy as jnp
from jax import lax
from jax.experimental import pallas as pl
from jax.experimental.pallas import tpu as pltpu
from jax.experimental.pallas import tpu_sc as plsc

_D = 80
_HE = 64
_SAW = 128
_TILE = 4096
_IDXC = 128


def _sc_gather(table, idx):
    _, d = table.shape
    b = idx.shape[0]
    info = plsc.get_sparse_core_info()
    nw = info.num_cores * info.num_subcores
    b_per_w = b // nw
    nchunk = b_per_w // _IDXC
    mesh = plsc.VectorSubcoreMesh(core_axis_name="c", subcore_axis_name="s")

    @functools.partial(
        pl.kernel,
        mesh=mesh,
        out_type=jax.ShapeDtypeStruct((b, d), jnp.float32),
        scratch_types=[
            pltpu.VMEM((nchunk, _IDXC), jnp.int32),
            pltpu.VMEM((nchunk, _IDXC, d), jnp.float32),
            pltpu.SemaphoreType.DMA,
        ],
        compiler_params=pltpu.CompilerParams(use_tc_tiling_on_sc=False),
    )
    def k(table_hbm, idx_hbm, out_hbm, idx_v, rows_v, sem):
        wid = lax.axis_index("s") * info.num_cores + lax.axis_index("c")
        base = wid * b_per_w
        for j in range(nchunk):
            pltpu.sync_copy(idx_hbm.at[pl.ds(base + j * _IDXC, _IDXC)],
                            idx_v.at[j])
        copies = [
            pltpu.async_copy(table_hbm.at[idx_v.at[j]], rows_v.at[j], sem)
            for j in range(nchunk)
        ]
        for c in copies:
            c.wait()
        for j in range(nchunk):
            pltpu.sync_copy(rows_v.at[j],
                            out_hbm.at[pl.ds(base + j * _IDXC, _IDXC)])

    return k(table, idx)


def _tc_body(z_ref, tv_ref, ball_ref,
             ew1_ref, wn1_ref, wr1_ref, wr2_ref, ew2_ref, wn2_ref, wn3_ref,
             ww1_ref, ww2_ref, ww3_ref, wi1_ref, wt1_ref, wi2_ref, wt2_ref,
             wi3_ref, wt3_ref, wo1_ref, wo2_ref, wo3_ref,
             hout_ref, cur_ref,
             wp1_s, wp2_s, wd_s, we_s, wf_s,
             bp1_s, bp2_s, bn3_s, ba_s, bb_s, bc_s, bd_s, be_s, bf_s,
             bg_s, bh_s, bi_s):
    f32 = jnp.float32

    @pl.when(pl.program_id(0) == 0)
    def _prep():
        bcol = jnp.transpose(ball_ref[...])

        wp1_s[0:64, :] = ew1_ref[0]
        wp1_s[64:128, :] = ew1_ref[1]
        wp1_s[128:208, :] = wn1_ref[...]
        wp1_s[208:210, :] = wr1_ref[0:2, :]
        wp1_s[210:212, :] = wr2_ref[0:2, :]
        bp1_s[0:128, :] = bcol[0:128]
        bp1_s[128:208, :] = bcol[128:208]
        bp1_s[208:210, :] = bcol[208:210]
        bp1_s[210:212, :] = bcol[272:274]

        wp2_s[...] = jnp.zeros(wp2_s.shape, f32)
        wp2_s[0:80, 0:64] = ew2_ref[0]
        wp2_s[80:160, 64:128] = ew2_ref[1]
        wp2_s[160:200, 128:208] = wn2_ref[...]
        bp2_s[0:160, :] = bcol[336:496]
        bp2_s[160:200, :] = bcol[496:536]
        bn3_s[...] = bcol[1720:1721]

        ba_s[...] = bcol[536:664]
        bb_s[...] = bcol[664:792]
        bc_s[...] = bcol[792:856]

        wd_s[0:128, :] = wi1_ref[...]
        wd_s[128:256, :] = wt1_ref[...]
        bd_s[0:128, :] = bcol[856:984]
        bd_s[128:256, :] = bcol[984:1112]

        we_s[...] = jnp.zeros(we_s.shape, f32)
        we_s[0:64, 0:128] = wi2_ref[...]
        we_s[64:192, 128:256] = wt2_ref[...]
        be_s[0:64, :] = bcol[1112:1176]
        be_s[64:192, :] = bcol[1176:1304]

        wf_s[...] = jnp.zeros(wf_s.shape, f32)
        wf_s[0:80, 64:192] = wt3_ref[...]
        wf_s[80:81, 0:64] = wi3_ref[...]
        bf_s[0:80, :] = bcol[1304:1384]
        bf_s[80:81, :] = bcol[1721:1722]

        bg_s[...] = bcol[1384:1512]
        bh_s[...] = bcol[1512:1640]
        bi_s[...] = bcol[1640:1720]

    def mm(w, b_ref, x):
        y = lax.dot_general(w, x, (((1,), (0,)), ((), ())),
                            preferred_element_type=f32)
        return y + b_ref[...]

    sig = jax.nn.sigmoid
    n = z_ref.shape[0]
    h = jnp.concatenate(
        [jnp.transpose(z_ref[...]), jnp.transpose(tv_ref[...])], axis=0)
    cur_acc = jnp.zeros((1, n), f32)
    ctx_acc = jnp.zeros((1, n), f32)
    for _ in range(2):
        p1 = mm(wp1_s[...], bp1_s, h)
        r = jnp.maximum(p1[:208], 0.0)
        p2 = jnp.maximum(mm(wp2_s[...], bp2_s, r), 0.0)
        novelty = sig(mm(wn3_ref[...], bn3_s, p2[160:200]))
        d1 = p1[208:209] - p1[209:210]
        d2 = p1[210:211] - p1[211:212]
        m0 = (sig(d1) + sig(d2)) * 0.5
        m1 = (sig(-d1) + sig(-d2)) * 0.5
        h_next = p2[:80] * m0 + p2[80:160] * m1
        diff = h_next - h
        fe = jnp.sqrt(jnp.sum(diff * diff, axis=0, keepdims=True))
        err = sig(fe - 0.5)
        cur_acc = cur_acc + jnp.clip(0.4 * err + 0.6 * novelty, 0.1, 0.9)
        ctx_acc = ctx_acc + (m0 + m1)
        h = h_next
    ctx = ctx_acc * 0.25

    w = jnp.maximum(mm(ww1_ref[...], ba_s,
                       jnp.concatenate([h, ctx * h], axis=0)), 0.0)
    w = jnp.maximum(mm(ww2_ref[...], bb_s, w), 0.0)
    why = mm(ww3_ref[...], bc_s, w)
    qt = jnp.maximum(mm(wd_s[...], bd_s,
                        jnp.concatenate([h, why], axis=0)), 0.0)
    qt = jnp.maximum(mm(we_s[...], be_s, qt), 0.0)
    f = mm(wf_s[...], bf_s, qt)
    blended = h + sig(f[80:81]) * f[:80]
    o = jnp.maximum(mm(wo1_ref[...], bg_s,
                       jnp.concatenate([h, blended], axis=0)), 0.0)
    o = jnp.maximum(mm(wo2_ref[...], bh_s, o), 0.0)
    h_final = 0.7 * h + 0.3 * mm(wo3_ref[...], bi_s, o)
    hout_ref[...] = jnp.transpose(h_final)
    cur_ref[...] = jnp.transpose(cur_acc * 0.5)


def _tc_forward(z, tv, ball, raw):
    b = z.shape[0]
    grid = (b // _TILE,)
    f32 = jnp.float32

    def rowspec(cols):
        return pl.BlockSpec((_TILE, cols), lambda i: (i, 0))

    def fullspec(arr):
        return pl.BlockSpec(arr.shape, lambda i: (0,) * arr.ndim)

    in_specs = [rowspec(z.shape[1]), rowspec(tv.shape[1]), fullspec(ball)]
    in_specs += [fullspec(x) for x in raw]
    return pl.pallas_call(
        _tc_body,
        grid=grid,
        in_specs=in_specs,
        out_specs=[rowspec(_D), rowspec(1)],
        out_shape=[
            jax.ShapeDtypeStruct((b, _D), f32),
            jax.ShapeDtypeStruct((b, 1), f32),
        ],
        scratch_shapes=[
            pltpu.VMEM((212, 80), f32),
            pltpu.VMEM((200, 208), f32),
            pltpu.VMEM((256, 144), f32),
            pltpu.VMEM((192, 256), f32),
            pltpu.VMEM((81, 192), f32),
            pltpu.VMEM((212, 1), f32), pltpu.VMEM((200, 1), f32),
            pltpu.VMEM((1, 1), f32), pltpu.VMEM((128, 1), f32),
            pltpu.VMEM((128, 1), f32), pltpu.VMEM((64, 1), f32),
            pltpu.VMEM((256, 1), f32), pltpu.VMEM((192, 1), f32),
            pltpu.VMEM((81, 1), f32), pltpu.VMEM((128, 1), f32),
            pltpu.VMEM((128, 1), f32), pltpu.VMEM((80, 1), f32),
        ],
        compiler_params=pltpu.CompilerParams(
            dimension_semantics=("arbitrary",),
        ),
    )(z, tv, ball, *raw)


def kernel(z, task_ids, params):
    p = params
    tv = _sc_gather(p['task_emb'].astype(jnp.float32),
                    task_ids.astype(jnp.int32))
    ball = jnp.concatenate(
        [p[k].reshape(1, -1).astype(jnp.float32) for k in
         ('eb1', 'bn1', 'br1', 'br2', 'eb2', 'bn2', 'bw1', 'bw2', 'bw3',
          'bi1', 'bt1', 'bi2', 'bt2', 'bt3', 'bo1', 'bo2', 'bo3', 'bn3',
          'bi3')], axis=1)
    raw = [p[k] for k in
           ('eW1', 'Wn1', 'Wr1', 'Wr2', 'eW2', 'Wn2', 'Wn3', 'Ww1', 'Ww2',
            'Ww3', 'Wi1', 'Wt1', 'Wi2', 'Wt2', 'Wi3', 'Wt3', 'Wo1', 'Wo2',
            'Wo3')]
    h_final, avg_cur = _tc_forward(z, tv, ball, raw)
    return h_final, avg_cur


# speedup vs baseline: 1.0625x; 1.0576x over previous
import functools

import jax
import jax.numpy as jnp
from jax import lax
from jax.experimental import pallas as pl
from jax.experimental.pallas import tpu as pltpu
from jax.experimental.pallas import tpu_sc as plsc

_D = 80
_HE = 64
_SAW = 128
_TILE = 4096
_IDXC = 128


def _sc_gather(table, idx):
    _, d = table.shape
    b = idx.shape[0]
    info = plsc.get_sparse_core_info()
    nw = info.num_cores * info.num_subcores
    b_per_w = b // nw
    nchunk = b_per_w // _IDXC
    mesh = plsc.VectorSubcoreMesh(core_axis_name="c", subcore_axis_name="s")

    @functools.partial(
        pl.kernel,
        mesh=mesh,
        out_type=jax.ShapeDtypeStruct((b, d), jnp.float32),
        scratch_types=[
            pltpu.VMEM((nchunk, _IDXC), jnp.int32),
            pltpu.VMEM((nchunk, _IDXC, d), jnp.float32),
            pltpu.SemaphoreType.DMA,
        ],
    )
    def k(table_hbm, idx_hbm, out_hbm, idx_v, rows_v, sem):
        wid = lax.axis_index("s") * info.num_cores + lax.axis_index("c")
        base = wid * b_per_w
        for j in range(nchunk):
            pltpu.sync_copy(idx_hbm.at[pl.ds(base + j * _IDXC, _IDXC)],
                            idx_v.at[j])
        copies = [
            pltpu.async_copy(table_hbm.at[idx_v.at[j]], rows_v.at[j], sem)
            for j in range(nchunk)
        ]
        for c in copies:
            c.wait()
        for j in range(nchunk):
            pltpu.sync_copy(rows_v.at[j],
                            out_hbm.at[pl.ds(base + j * _IDXC, _IDXC)])

    return k(table, idx)


def _tc_body(z_ref, tv_ref, ball_ref,
             ew1_ref, wn1_ref, wr1_ref, wr2_ref, ew2_ref, wn2_ref, wn3_ref,
             ww1_ref, ww2_ref, ww3_ref, wi1_ref, wt1_ref, wi2_ref, wt2_ref,
             wi3_ref, wt3_ref, wo1_ref, wo2_ref, wo3_ref,
             hout_ref, cur_ref,
             wp1_s, wp2_s, wd_s, we_s, wf_s,
             bp1_s, bp2_s, bn3_s, ba_s, bb_s, bc_s, bd_s, be_s, bf_s,
             bg_s, bh_s, bi_s):
    f32 = jnp.float32

    @pl.when(pl.program_id(0) == 0)
    def _prep():
        bcol = jnp.transpose(ball_ref[...])

        wp1_s[0:64, :] = ew1_ref[0]
        wp1_s[64:128, :] = ew1_ref[1]
        wp1_s[128:208, :] = wn1_ref[...]
        wp1_s[208:210, :] = wr1_ref[0:2, :]
        wp1_s[210:212, :] = wr2_ref[0:2, :]
        bp1_s[0:128, :] = bcol[0:128]
        bp1_s[128:208, :] = bcol[128:208]
        bp1_s[208:210, :] = bcol[208:210]
        bp1_s[210:212, :] = bcol[272:274]

        wp2_s[...] = jnp.zeros(wp2_s.shape, f32)
        wp2_s[0:80, 0:64] = ew2_ref[0]
        wp2_s[80:160, 64:128] = ew2_ref[1]
        wp2_s[160:200, 128:208] = wn2_ref[...]
        bp2_s[0:160, :] = bcol[336:496]
        bp2_s[160:200, :] = bcol[496:536]
        bn3_s[...] = bcol[1720:1721]

        ba_s[...] = bcol[536:664]
        bb_s[...] = bcol[664:792]
        bc_s[...] = bcol[792:856]

        wd_s[0:128, :] = wi1_ref[...]
        wd_s[128:256, :] = wt1_ref[...]
        bd_s[0:128, :] = bcol[856:984]
        bd_s[128:256, :] = bcol[984:1112]

        we_s[...] = jnp.zeros(we_s.shape, f32)
        we_s[0:64, 0:128] = wi2_ref[...]
        we_s[64:192, 128:256] = wt2_ref[...]
        be_s[0:64, :] = bcol[1112:1176]
        be_s[64:192, :] = bcol[1176:1304]

        wf_s[...] = jnp.zeros(wf_s.shape, f32)
        wf_s[0:80, 64:192] = wt3_ref[...]
        wf_s[80:81, 0:64] = wi3_ref[...]
        bf_s[0:80, :] = bcol[1304:1384]
        bf_s[80:81, :] = bcol[1721:1722]

        bg_s[...] = bcol[1384:1512]
        bh_s[...] = bcol[1512:1640]
        bi_s[...] = bcol[1640:1720]

    def mm(w, b_ref, x):
        y = lax.dot_general(w, x, (((1,), (0,)), ((), ())),
                            preferred_element_type=f32)
        return y + b_ref[...]

    sig = jax.nn.sigmoid
    n = z_ref.shape[0]
    h = jnp.concatenate(
        [jnp.transpose(z_ref[...]), jnp.transpose(tv_ref[:, 0:16])], axis=0)
    cur_acc = jnp.zeros((1, n), f32)
    ctx_acc = jnp.zeros((1, n), f32)
    for _ in range(2):
        p1 = mm(wp1_s[...], bp1_s, h)
        r = jnp.maximum(p1[:208], 0.0)
        p2 = jnp.maximum(mm(wp2_s[...], bp2_s, r), 0.0)
        novelty = sig(mm(wn3_ref[...], bn3_s, p2[160:200]))
        d1 = p1[208:209] - p1[209:210]
        d2 = p1[210:211] - p1[211:212]
        m0 = (sig(d1) + sig(d2)) * 0.5
        m1 = (sig(-d1) + sig(-d2)) * 0.5
        h_next = p2[:80] * m0 + p2[80:160] * m1
        diff = h_next - h
        fe = jnp.sqrt(jnp.sum(diff * diff, axis=0, keepdims=True))
        err = sig(fe - 0.5)
        cur_acc = cur_acc + jnp.clip(0.4 * err + 0.6 * novelty, 0.1, 0.9)
        ctx_acc = ctx_acc + (m0 + m1)
        h = h_next
    ctx = ctx_acc * 0.25

    w = jnp.maximum(mm(ww1_ref[...], ba_s,
                       jnp.concatenate([h, ctx * h], axis=0)), 0.0)
    w = jnp.maximum(mm(ww2_ref[...], bb_s, w), 0.0)
    why = mm(ww3_ref[...], bc_s, w)
    qt = jnp.maximum(mm(wd_s[...], bd_s,
                        jnp.concatenate([h, why], axis=0)), 0.0)
    qt = jnp.maximum(mm(we_s[...], be_s, qt), 0.0)
    f = mm(wf_s[...], bf_s, qt)
    blended = h + sig(f[80:81]) * f[:80]
    o = jnp.maximum(mm(wo1_ref[...], bg_s,
                       jnp.concatenate([h, blended], axis=0)), 0.0)
    o = jnp.maximum(mm(wo2_ref[...], bh_s, o), 0.0)
    h_final = 0.7 * h + 0.3 * mm(wo3_ref[...], bi_s, o)
    hout_ref[...] = jnp.transpose(h_final)
    cur_ref[...] = jnp.transpose(cur_acc * 0.5)


def _tc_forward(z, tv, ball, raw):
    b = z.shape[0]
    grid = (b // _TILE,)
    f32 = jnp.float32

    def rowspec(cols):
        return pl.BlockSpec((_TILE, cols), lambda i: (i, 0))

    def fullspec(arr):
        return pl.BlockSpec(arr.shape, lambda i: (0,) * arr.ndim)

    in_specs = [rowspec(z.shape[1]), rowspec(tv.shape[1]), fullspec(ball)]
    in_specs += [fullspec(x) for x in raw]
    return pl.pallas_call(
        _tc_body,
        grid=grid,
        in_specs=in_specs,
        out_specs=[rowspec(_D), rowspec(1)],
        out_shape=[
            jax.ShapeDtypeStruct((b, _D), f32),
            jax.ShapeDtypeStruct((b, 1), f32),
        ],
        scratch_shapes=[
            pltpu.VMEM((212, 80), f32),
            pltpu.VMEM((200, 208), f32),
            pltpu.VMEM((256, 144), f32),
            pltpu.VMEM((192, 256), f32),
            pltpu.VMEM((81, 192), f32),
            pltpu.VMEM((212, 1), f32), pltpu.VMEM((200, 1), f32),
            pltpu.VMEM((1, 1), f32), pltpu.VMEM((128, 1), f32),
            pltpu.VMEM((128, 1), f32), pltpu.VMEM((64, 1), f32),
            pltpu.VMEM((256, 1), f32), pltpu.VMEM((192, 1), f32),
            pltpu.VMEM((81, 1), f32), pltpu.VMEM((128, 1), f32),
            pltpu.VMEM((128, 1), f32), pltpu.VMEM((80, 1), f32),
        ],
        compiler_params=pltpu.CompilerParams(
            dimension_semantics=("arbitrary",),
        ),
    )(z, tv, ball, *raw)


def kernel(z, task_ids, params):
    p = params
    table_p = jnp.pad(p['task_emb'].astype(jnp.float32), ((0, 0), (0, 112)))
    tv = _sc_gather(table_p, task_ids.astype(jnp.int32))
    ball = jnp.concatenate(
        [p[k].reshape(1, -1).astype(jnp.float32) for k in
         ('eb1', 'bn1', 'br1', 'br2', 'eb2', 'bn2', 'bw1', 'bw2', 'bw3',
          'bi1', 'bt1', 'bi2', 'bt2', 'bt3', 'bo1', 'bo2', 'bo3', 'bn3',
          'bi3')], axis=1)
    raw = [p[k] for k in
           ('eW1', 'Wn1', 'Wr1', 'Wr2', 'eW2', 'Wn2', 'Wn3', 'Ww1', 'Ww2',
            'Ww3', 'Wi1', 'Wt1', 'Wi2', 'Wt2', 'Wi3', 'Wt3', 'Wo1', 'Wo2',
            'Wo3')]
    h_final, avg_cur = _tc_forward(z, tv, ball, raw)
    return h_final, avg_cur


# speedup vs baseline: 1.3285x; 1.2504x over previous
import functools

import jax
import jax.numpy as jnp
from jax import lax
from jax.experimental import pallas as pl
from jax.experimental.pallas import tpu as pltpu
from jax.experimental.pallas import tpu_sc as plsc

_D = 80
_HE = 64
_SAW = 128
_TILE = 4096
_IDXC = 128


def _sc_gather(table, idx):
    _, d = table.shape
    b = idx.shape[0]
    info = plsc.get_sparse_core_info()
    nw = info.num_cores * info.num_subcores
    b_per_w = b // nw
    nchunk = b_per_w // _IDXC
    mesh = plsc.VectorSubcoreMesh(core_axis_name="c", subcore_axis_name="s")

    @functools.partial(
        pl.kernel,
        mesh=mesh,
        out_type=jax.ShapeDtypeStruct((b, d), jnp.float32),
        scratch_types=[
            pltpu.VMEM((nchunk, _IDXC), jnp.int32),
            pltpu.VMEM((nchunk, _IDXC, d), jnp.float32),
            pltpu.SemaphoreType.DMA,
        ],
    )
    def k(table_hbm, idx_hbm, out_hbm, idx_v, rows_v, sem):
        wid = lax.axis_index("s") * info.num_cores + lax.axis_index("c")
        base = wid * b_per_w
        for j in range(nchunk):
            pltpu.sync_copy(idx_hbm.at[pl.ds(base + j * _IDXC, _IDXC)],
                            idx_v.at[j])
        copies = [
            pltpu.async_copy(table_hbm.at[idx_v.at[j]], rows_v.at[j], sem)
            for j in range(nchunk)
        ]
        for c in copies:
            c.wait()
        for j in range(nchunk):
            pltpu.sync_copy(rows_v.at[j],
                            out_hbm.at[pl.ds(base + j * _IDXC, _IDXC)])

    return k(table, idx)


def _tc_body(z_ref, tv_ref, ball_ref,
             ew1_ref, wn1_ref, wr1_ref, wr2_ref, ew2_ref, wn2_ref, wn3_ref,
             ww1_ref, ww2_ref, ww3_ref, wi1_ref, wt1_ref, wi2_ref, wt2_ref,
             wi3_ref, wt3_ref, wo1_ref, wo2_ref, wo3_ref,
             hout_ref, cur_ref,
             wp1_s, wp2_s, wd_s, we_s, wf_s,
             bp1_s, bp2_s, bn3_s, ba_s, bb_s, bc_s, bd_s, be_s, bf_s,
             bg_s, bh_s, bi_s):
    f32 = jnp.float32

    @pl.when(pl.program_id(0) == 0)
    def _prep():
        bcol = jnp.transpose(ball_ref[...])

        wp1_s[0:64, :] = ew1_ref[0]
        wp1_s[64:128, :] = ew1_ref[1]
        wp1_s[128:208, :] = wn1_ref[...]
        wp1_s[208:210, :] = wr1_ref[0:2, :]
        wp1_s[210:212, :] = wr2_ref[0:2, :]
        bp1_s[0:128, :] = bcol[0:128]
        bp1_s[128:208, :] = bcol[128:208]
        bp1_s[208:210, :] = bcol[208:210]
        bp1_s[210:212, :] = bcol[272:274]

        wp2_s[...] = jnp.zeros(wp2_s.shape, f32)
        wp2_s[0:80, 0:64] = ew2_ref[0]
        wp2_s[80:160, 64:128] = ew2_ref[1]
        wp2_s[160:200, 128:208] = wn2_ref[...]
        bp2_s[0:160, :] = bcol[336:496]
        bp2_s[160:200, :] = bcol[496:536]
        bn3_s[...] = bcol[1720:1721]

        ba_s[...] = bcol[536:664]
        bb_s[...] = bcol[664:792]
        bc_s[...] = bcol[792:856]

        wd_s[0:128, :] = wi1_ref[...]
        wd_s[128:256, :] = wt1_ref[...]
        bd_s[0:128, :] = bcol[856:984]
        bd_s[128:256, :] = bcol[984:1112]

        we_s[...] = jnp.zeros(we_s.shape, f32)
        we_s[0:64, 0:128] = wi2_ref[...]
        we_s[64:192, 128:256] = wt2_ref[...]
        be_s[0:64, :] = bcol[1112:1176]
        be_s[64:192, :] = bcol[1176:1304]

        wf_s[...] = jnp.zeros(wf_s.shape, f32)
        wf_s[0:80, 64:192] = wt3_ref[...]
        wf_s[80:81, 0:64] = wi3_ref[...]
        bf_s[0:80, :] = bcol[1304:1384]
        bf_s[80:81, :] = bcol[1721:1722]

        bg_s[...] = bcol[1384:1512]
        bh_s[...] = bcol[1512:1640]
        bi_s[...] = bcol[1640:1720]

    def mm(w, b_ref, x):
        y = lax.dot_general(w, x, (((1,), (0,)), ((), ())),
                            preferred_element_type=f32)
        return y + b_ref[...]

    sig = jax.nn.sigmoid
    n = z_ref.shape[0]
    h = jnp.concatenate(
        [jnp.transpose(z_ref[...]), jnp.transpose(tv_ref[:, 0:16])], axis=0)
    cur_acc = jnp.zeros((1, n), f32)
    ctx_acc = jnp.zeros((1, n), f32)
    for _ in range(2):
        p1 = mm(wp1_s[...], bp1_s, h)
        r = jnp.maximum(p1[:208], 0.0)
        p2 = jnp.maximum(mm(wp2_s[...], bp2_s, r), 0.0)
        novelty = sig(mm(wn3_ref[...], bn3_s, p2[160:200]))
        d1 = p1[208:209] - p1[209:210]
        d2 = p1[210:211] - p1[211:212]
        m0 = (sig(d1) + sig(d2)) * 0.5
        m1 = (sig(-d1) + sig(-d2)) * 0.5
        h_next = p2[:80] * m0 + p2[80:160] * m1
        diff = h_next - h
        fe = jnp.sqrt(jnp.sum(diff * diff, axis=0, keepdims=True))
        err = sig(fe - 0.5)
        cur_acc = cur_acc + jnp.clip(0.4 * err + 0.6 * novelty, 0.1, 0.9)
        ctx_acc = ctx_acc + (m0 + m1)
        h = h_next
    ctx = ctx_acc * 0.25

    w = jnp.maximum(mm(ww1_ref[...], ba_s,
                       jnp.concatenate([h, ctx * h], axis=0)), 0.0)
    w = jnp.maximum(mm(ww2_ref[...], bb_s, w), 0.0)
    why = mm(ww3_ref[...], bc_s, w)
    qt = jnp.maximum(mm(wd_s[...], bd_s,
                        jnp.concatenate([h, why], axis=0)), 0.0)
    qt = jnp.maximum(mm(we_s[...], be_s, qt), 0.0)
    f = mm(wf_s[...], bf_s, qt)
    blended = h + sig(f[80:81]) * f[:80]
    o = jnp.maximum(mm(wo1_ref[...], bg_s,
                       jnp.concatenate([h, blended], axis=0)), 0.0)
    o = jnp.maximum(mm(wo2_ref[...], bh_s, o), 0.0)
    h_final = 0.7 * h + 0.3 * mm(wo3_ref[...], bi_s, o)
    hout_ref[...] = h_final
    cur_ref[...] = cur_acc * 0.5


def _tc_forward(z, tv, ball, raw):
    b = z.shape[0]
    grid = (b // _TILE,)
    f32 = jnp.float32

    def rowspec(cols):
        return pl.BlockSpec((_TILE, cols), lambda i: (i, 0))

    def fullspec(arr):
        return pl.BlockSpec(arr.shape, lambda i: (0,) * arr.ndim)

    in_specs = [rowspec(z.shape[1]), rowspec(tv.shape[1]), fullspec(ball)]
    in_specs += [fullspec(x) for x in raw]
    return pl.pallas_call(
        _tc_body,
        grid=grid,
        in_specs=in_specs,
        out_specs=[pl.BlockSpec((_D, _TILE), lambda i: (0, i)),
                   pl.BlockSpec((1, _TILE), lambda i: (0, i))],
        out_shape=[
            jax.ShapeDtypeStruct((_D, b), f32),
            jax.ShapeDtypeStruct((1, b), f32),
        ],
        scratch_shapes=[
            pltpu.VMEM((212, 80), f32),
            pltpu.VMEM((200, 208), f32),
            pltpu.VMEM((256, 144), f32),
            pltpu.VMEM((192, 256), f32),
            pltpu.VMEM((81, 192), f32),
            pltpu.VMEM((212, 1), f32), pltpu.VMEM((200, 1), f32),
            pltpu.VMEM((1, 1), f32), pltpu.VMEM((128, 1), f32),
            pltpu.VMEM((128, 1), f32), pltpu.VMEM((64, 1), f32),
            pltpu.VMEM((256, 1), f32), pltpu.VMEM((192, 1), f32),
            pltpu.VMEM((81, 1), f32), pltpu.VMEM((128, 1), f32),
            pltpu.VMEM((128, 1), f32), pltpu.VMEM((80, 1), f32),
        ],
        compiler_params=pltpu.CompilerParams(
            dimension_semantics=("arbitrary",),
        ),
    )(z, tv, ball, *raw)


def kernel(z, task_ids, params):
    p = params
    table_p = jnp.pad(p['task_emb'].astype(jnp.float32), ((0, 0), (0, 112)))
    tv = _sc_gather(table_p, task_ids.astype(jnp.int32))
    ball = jnp.concatenate(
        [p[k].reshape(1, -1).astype(jnp.float32) for k in
         ('eb1', 'bn1', 'br1', 'br2', 'eb2', 'bn2', 'bw1', 'bw2', 'bw3',
          'bi1', 'bt1', 'bi2', 'bt2', 'bt3', 'bo1', 'bo2', 'bo3', 'bn3',
          'bi3')], axis=1)
    raw = [p[k] for k in
           ('eW1', 'Wn1', 'Wr1', 'Wr2', 'eW2', 'Wn2', 'Wn3', 'Ww1', 'Ww2',
            'Ww3', 'Wi1', 'Wt1', 'Wi2', 'Wt2', 'Wi3', 'Wt3', 'Wo1', 'Wo2',
            'Wo3')]
    h_t, cur_t = _tc_forward(z, tv, ball, raw)
    return jnp.transpose(h_t), jnp.transpose(cur_t)


# speedup vs baseline: 1.4646x; 1.1025x over previous
import functools

import jax
import jax.numpy as jnp
from jax import lax
from jax.experimental import pallas as pl
from jax.experimental.pallas import tpu as pltpu
from jax.experimental.pallas import tpu_sc as plsc

_D = 80
_HE = 64
_SAW = 128
_TILE = 4096
_IDXC = 128


def _sc_gather(table, idx):
    _, d = table.shape
    b = idx.shape[0]
    info = plsc.get_sparse_core_info()
    nw = info.num_cores * info.num_subcores
    b_per_w = b // nw
    nchunk = b_per_w // _IDXC
    mesh = plsc.VectorSubcoreMesh(core_axis_name="c", subcore_axis_name="s")

    @functools.partial(
        pl.kernel,
        mesh=mesh,
        out_type=jax.ShapeDtypeStruct((b, d), jnp.float32),
        scratch_types=[
            pltpu.VMEM((nchunk, _IDXC), jnp.int32),
            pltpu.VMEM((nchunk, _IDXC, d), jnp.float32),
            pltpu.SemaphoreType.DMA,
        ],
    )
    def k(table_hbm, idx_hbm, out_hbm, idx_v, rows_v, sem):
        wid = lax.axis_index("s") * info.num_cores + lax.axis_index("c")
        base = wid * b_per_w
        for j in range(nchunk):
            pltpu.sync_copy(idx_hbm.at[pl.ds(base + j * _IDXC, _IDXC)],
                            idx_v.at[j])
        copies = [
            pltpu.async_copy(table_hbm.at[idx_v.at[j]], rows_v.at[j], sem)
            for j in range(nchunk)
        ]
        for c in copies:
            c.wait()
        for j in range(nchunk):
            pltpu.sync_copy(rows_v.at[j],
                            out_hbm.at[pl.ds(base + j * _IDXC, _IDXC)])

    return k(table, idx)


def _tc_body(z_ref, tv_ref, ball_ref,
             ew1_ref, wn1_ref, wr1_ref, wr2_ref, ew2_ref, wn2_ref, wn3_ref,
             ww1_ref, ww2_ref, ww3_ref, wi1_ref, wt1_ref, wi2_ref, wt2_ref,
             wi3_ref, wt3_ref, wo1_ref, wo2_ref, wo3_ref,
             hout_ref, cur_ref,
             wp1_s, wp2_s, wd_s, we_s, wf_s,
             bp1_s, bp2_s, bn3_s, ba_s, bb_s, bc_s, bd_s, be_s, bf_s,
             bg_s, bh_s, bi_s):
    f32 = jnp.float32

    @pl.when(pl.program_id(0) == 0)
    def _prep():
        bcol = jnp.transpose(ball_ref[...])

        wp1_s[0:64, :] = ew1_ref[0]
        wp1_s[64:128, :] = ew1_ref[1]
        wp1_s[128:208, :] = wn1_ref[...]
        wp1_s[208:210, :] = wr1_ref[0:2, :]
        wp1_s[210:212, :] = wr2_ref[0:2, :]
        bp1_s[0:128, :] = bcol[0:128]
        bp1_s[128:208, :] = bcol[128:208]
        bp1_s[208:210, :] = bcol[208:210]
        bp1_s[210:212, :] = bcol[272:274]

        wp2_s[...] = jnp.zeros(wp2_s.shape, f32)
        wp2_s[0:80, 0:64] = ew2_ref[0]
        wp2_s[80:160, 64:128] = ew2_ref[1]
        wp2_s[160:200, 128:208] = wn2_ref[...]
        bp2_s[0:160, :] = bcol[336:496]
        bp2_s[160:200, :] = bcol[496:536]
        bn3_s[...] = bcol[1720:1721]

        ba_s[...] = bcol[536:664]
        bb_s[...] = bcol[664:792]
        bc_s[...] = bcol[792:856]

        wd_s[0:128, :] = wi1_ref[...]
        wd_s[128:256, :] = wt1_ref[...]
        bd_s[0:128, :] = bcol[856:984]
        bd_s[128:256, :] = bcol[984:1112]

        we_s[...] = jnp.zeros(we_s.shape, f32)
        we_s[0:64, 0:128] = wi2_ref[...]
        we_s[64:192, 128:256] = wt2_ref[...]
        be_s[0:64, :] = bcol[1112:1176]
        be_s[64:192, :] = bcol[1176:1304]

        wf_s[...] = jnp.zeros(wf_s.shape, f32)
        wf_s[0:80, 64:192] = wt3_ref[...]
        wf_s[80:81, 0:64] = wi3_ref[...]
        bf_s[0:80, :] = bcol[1304:1384]
        bf_s[80:81, :] = bcol[1721:1722]

        bg_s[...] = bcol[1384:1512]
        bh_s[...] = bcol[1512:1640]
        bi_s[...] = bcol[1640:1720]

    def mm(w, b_ref, x):
        y = lax.dot_general(w, x, (((1,), (0,)), ((), ())),
                            preferred_element_type=f32)
        return y + b_ref[...]

    sig = jax.nn.sigmoid
    n = z_ref.shape[1]
    h = jnp.concatenate(
        [z_ref[...], jnp.transpose(tv_ref[:, 0:16])], axis=0)
    cur_acc = jnp.zeros((1, n), f32)
    ctx_acc = jnp.zeros((1, n), f32)
    for _ in range(2):
        p1 = mm(wp1_s[...], bp1_s, h)
        r = jnp.maximum(p1[:208], 0.0)
        p2 = jnp.maximum(mm(wp2_s[...], bp2_s, r), 0.0)
        novelty = sig(mm(wn3_ref[...], bn3_s, p2[160:200]))
        d1 = p1[208:209] - p1[209:210]
        d2 = p1[210:211] - p1[211:212]
        m0 = (sig(d1) + sig(d2)) * 0.5
        m1 = (sig(-d1) + sig(-d2)) * 0.5
        h_next = p2[:80] * m0 + p2[80:160] * m1
        diff = h_next - h
        fe = jnp.sqrt(jnp.sum(diff * diff, axis=0, keepdims=True))
        err = sig(fe - 0.5)
        cur_acc = cur_acc + jnp.clip(0.4 * err + 0.6 * novelty, 0.1, 0.9)
        ctx_acc = ctx_acc + (m0 + m1)
        h = h_next
    ctx = ctx_acc * 0.25

    w = jnp.maximum(mm(ww1_ref[...], ba_s,
                       jnp.concatenate([h, ctx * h], axis=0)), 0.0)
    w = jnp.maximum(mm(ww2_ref[...], bb_s, w), 0.0)
    why = mm(ww3_ref[...], bc_s, w)
    qt = jnp.maximum(mm(wd_s[...], bd_s,
                        jnp.concatenate([h, why], axis=0)), 0.0)
    qt = jnp.maximum(mm(we_s[...], be_s, qt), 0.0)
    f = mm(wf_s[...], bf_s, qt)
    blended = h + sig(f[80:81]) * f[:80]
    o = jnp.maximum(mm(wo1_ref[...], bg_s,
                       jnp.concatenate([h, blended], axis=0)), 0.0)
    o = jnp.maximum(mm(wo2_ref[...], bh_s, o), 0.0)
    h_final = 0.7 * h + 0.3 * mm(wo3_ref[...], bi_s, o)
    hout_ref[...] = h_final
    cur_ref[...] = cur_acc * 0.5


def _tc_forward(z, tv, ball, raw):
    b = z.shape[1]
    grid = (b // _TILE,)
    f32 = jnp.float32

    def rowspec(cols):
        return pl.BlockSpec((_TILE, cols), lambda i: (i, 0))

    def fullspec(arr):
        return pl.BlockSpec(arr.shape, lambda i: (0,) * arr.ndim)

    in_specs = [pl.BlockSpec((z.shape[0], _TILE), lambda i: (0, i)),
                rowspec(tv.shape[1]), fullspec(ball)]
    in_specs += [fullspec(x) for x in raw]
    return pl.pallas_call(
        _tc_body,
        grid=grid,
        in_specs=in_specs,
        out_specs=[pl.BlockSpec((_D, _TILE), lambda i: (0, i)),
                   pl.BlockSpec((1, _TILE), lambda i: (0, i))],
        out_shape=[
            jax.ShapeDtypeStruct((_D, b), f32),
            jax.ShapeDtypeStruct((1, b), f32),
        ],
        scratch_shapes=[
            pltpu.VMEM((212, 80), f32),
            pltpu.VMEM((200, 208), f32),
            pltpu.VMEM((256, 144), f32),
            pltpu.VMEM((192, 256), f32),
            pltpu.VMEM((81, 192), f32),
            pltpu.VMEM((212, 1), f32), pltpu.VMEM((200, 1), f32),
            pltpu.VMEM((1, 1), f32), pltpu.VMEM((128, 1), f32),
            pltpu.VMEM((128, 1), f32), pltpu.VMEM((64, 1), f32),
            pltpu.VMEM((256, 1), f32), pltpu.VMEM((192, 1), f32),
            pltpu.VMEM((81, 1), f32), pltpu.VMEM((128, 1), f32),
            pltpu.VMEM((128, 1), f32), pltpu.VMEM((80, 1), f32),
        ],
        compiler_params=pltpu.CompilerParams(
            dimension_semantics=("arbitrary",),
        ),
    )(z, tv, ball, *raw)


def kernel(z, task_ids, params):
    p = params
    table_p = jnp.pad(p['task_emb'].astype(jnp.float32), ((0, 0), (0, 112)))
    tv = _sc_gather(table_p, task_ids.astype(jnp.int32))
    ball = jnp.concatenate(
        [p[k].reshape(1, -1).astype(jnp.float32) for k in
         ('eb1', 'bn1', 'br1', 'br2', 'eb2', 'bn2', 'bw1', 'bw2', 'bw3',
          'bi1', 'bt1', 'bi2', 'bt2', 'bt3', 'bo1', 'bo2', 'bo3', 'bn3',
          'bi3')], axis=1)
    raw = [p[k] for k in
           ('eW1', 'Wn1', 'Wr1', 'Wr2', 'eW2', 'Wn2', 'Wn3', 'Ww1', 'Ww2',
            'Ww3', 'Wi1', 'Wt1', 'Wi2', 'Wt2', 'Wi3', 'Wt3', 'Wo1', 'Wo2',
            'Wo3')]
    h_t, cur_t = _tc_forward(jnp.transpose(z), tv, ball, raw)
    return jnp.transpose(h_t), jnp.transpose(cur_t)
